# baseline (speedup 1.0000x reference)
import functools

import jax
import jax.numpy as jnp
from jax import lax
from jax.experimental import pallas as pl
from jax.experimental.pallas import tpu as pltpu
from jax.experimental.pallas import tpu_sc as plsc

D_MODEL = 768
NUM_EXPERTS = 8
TOP_K = 2
MASK_W = TOP_K * NUM_EXPERTS


_TB = 1024
_NBUF = 6


def _logits_body(x_hbm, w_ref, out_ref, *scratch):
    n = out_ref.shape[1]
    nblk = n // _TB
    bufs = scratch[:_NBUF]
    sems = scratch[_NBUF]

    def start(i):
        pltpu.make_async_copy(
            x_hbm.at[pl.ds(i * _TB, _TB), :],
            bufs[i % _NBUF], sems.at[i % _NBUF]).start()

    for i in range(_NBUF):
        start(i)
    w = w_ref[...]
    for i in range(nblk):
        b = i % _NBUF
        pltpu.make_async_copy(
            x_hbm.at[pl.ds(i * _TB, _TB), :], bufs[b], sems.at[b]).wait()
        out_ref[:, pl.ds(i * _TB, _TB)] = lax.dot_general(
            w, bufs[b][...],
            dimension_numbers=(((1,), (1,)), ((), ())),
            preferred_element_type=jnp.float32)
        if i + _NBUF < nblk:
            start(i + _NBUF)


def _compute_logits_t(x, W):
    n = x.shape[0]
    return pl.pallas_call(
        _logits_body,
        in_specs=[pl.BlockSpec(memory_space=pltpu.HBM),
                  pl.BlockSpec((NUM_EXPERTS, D_MODEL), lambda: (0, 0))],
        out_specs=pl.BlockSpec((NUM_EXPERTS, n), lambda: (0, 0)),
        out_shape=jax.ShapeDtypeStruct((NUM_EXPERTS, n), jnp.float32),
        scratch_shapes=(
            [pltpu.VMEM((_TB, D_MODEL), jnp.float32) for _ in range(_NBUF)]
            + [pltpu.SemaphoreType.DMA((_NBUF,))]
        ),
    )(x, W)


@functools.lru_cache(maxsize=None)
def _make_router(n):
    info = plsc.get_sparse_core_info()
    nc, ns, lanes = info.num_cores, info.num_subcores, info.num_lanes
    nw = nc * ns
    tpw = n // nw
    mesh = plsc.VectorSubcoreMesh(core_axis_name="c", subcore_axis_name="s")

    @functools.partial(
        pl.kernel, mesh=mesh,
        compiler_params=pltpu.CompilerParams(
            needs_layout_passes=False, use_tc_tiling_on_sc=False),
        out_type=[
            jax.ShapeDtypeStruct((TOP_K * n,), jnp.float32),
            jax.ShapeDtypeStruct((TOP_K * n,), jnp.int32),
            jax.ShapeDtypeStruct((MASK_W * n,), jnp.float32),
        ],
        scratch_types=(
            [pltpu.VMEM((tpw,), jnp.float32) for _ in range(NUM_EXPERTS)]
            + [pltpu.VMEM((tpw,), jnp.float32) for _ in range(TOP_K)]
            + [pltpu.VMEM((tpw,), jnp.int32) for _ in range(TOP_K)]
            + [pltpu.VMEM((tpw,), jnp.float32) for _ in range(MASK_W)]
        ),
    )
    def router(logits_hbm, probs_hbm, idx_hbm, mask_hbm, *scratch):
        e_v = scratch[0:NUM_EXPERTS]
        p_v = scratch[NUM_EXPERTS:NUM_EXPERTS + TOP_K]
        ix_v = scratch[NUM_EXPERTS + TOP_K:NUM_EXPERTS + 2 * TOP_K]
        m_v = scratch[NUM_EXPERTS + 2 * TOP_K:]
        wid = lax.axis_index("s") * nc + lax.axis_index("c")
        base = wid * tpw
        for j in range(NUM_EXPERTS):
            pltpu.sync_copy(logits_hbm.at[pl.ds(j * n + base, tpw)], e_v[j])

        def chunk(i, carry):
            sl = pl.ds(i * lanes, lanes)
            e = [e_v[j][sl] for j in range(NUM_EXPERTS)]
            m1 = e[0]
            i1 = jnp.zeros((lanes,), jnp.int32)
            for j in range(1, NUM_EXPERTS):
                gt = e[j] > m1
                m1 = jnp.where(gt, e[j], m1)
                i1 = jnp.where(gt, j, i1)
            m2 = jnp.full((lanes,), -3e38, jnp.float32)
            i2 = jnp.zeros((lanes,), jnp.int32)
            for j in range(NUM_EXPERTS):
                gt = (e[j] > m2) & (i1 != j)
                m2 = jnp.where(gt, e[j], m2)
                i2 = jnp.where(gt, j, i2)
            d = jnp.exp(m2 - m1)
            p1 = 1.0 / (1.0 + d)
            p2 = d * p1
            p_v[0][sl] = p1
            p_v[1][sl] = p2
            ix_v[0][sl] = i1
            ix_v[1][sl] = i2
            for k in range(NUM_EXPERTS):
                m_v[k][sl] = jnp.where(i1 == k, 1.0, 0.0)
                m_v[NUM_EXPERTS + k][sl] = jnp.where(i2 == k, 1.0, 0.0)
            return carry

        lax.fori_loop(0, tpw // lanes, chunk, 0)
        for r in range(TOP_K):
            pltpu.sync_copy(p_v[r], probs_hbm.at[pl.ds(r * n + base, tpw)])
            pltpu.sync_copy(ix_v[r], idx_hbm.at[pl.ds(r * n + base, tpw)])
        for k in range(MASK_W):
            pltpu.sync_copy(m_v[k], mask_hbm.at[pl.ds(k * n + base, tpw)])

    return router


def kernel(x, W):
    n = x.shape[0]
    logits_t = _compute_logits_t(x, W)
    probs_t, idx_t, mask_t = _make_router(n)(logits_t.reshape(-1))
    probs = probs_t.reshape(TOP_K, n).T
    idx = idx_t.reshape(TOP_K, n).T
    mask = mask_t.reshape(TOP_K, NUM_EXPERTS, n).transpose(2, 0, 1)
    return probs, idx, mask


# speedup vs baseline: 1.1078x; 1.1078x over previous
import functools

import jax
import jax.numpy as jnp
from jax import lax
from jax.experimental import pallas as pl
from jax.experimental.pallas import tpu as pltpu
from jax.experimental.pallas import tpu_sc as plsc

D_MODEL = 768
NUM_EXPERTS = 8
TOP_K = 2
MASK_W = TOP_K * NUM_EXPERTS


_TB = 1024
_NBUF = 6


def _logits_body(x_hbm, w_ref, out_ref, *scratch):
    n = x_hbm.shape[0]
    nblk = n // _TB
    bufs = scratch[:_NBUF]
    sems = scratch[_NBUF]

    def start(i):
        pltpu.make_async_copy(
            x_hbm.at[pl.ds(i * _TB, _TB), :],
            bufs[i % _NBUF], sems.at[i % _NBUF]).start()

    for i in range(_NBUF):
        start(i)
    w = w_ref[...]
    for i in range(nblk):
        b = i % _NBUF
        pltpu.make_async_copy(
            x_hbm.at[pl.ds(i * _TB, _TB), :], bufs[b], sems.at[b]).wait()
        r = lax.dot_general(
            w, bufs[b][...],
            dimension_numbers=(((1,), (1,)), ((), ())),
            preferred_element_type=jnp.float32)
        for j in range(NUM_EXPERTS):
            out_ref[pl.ds(j * n + i * _TB, _TB)] = r[j]
        if i + _NBUF < nblk:
            start(i + _NBUF)


def _compute_logits_t(x, W):
    n = x.shape[0]
    return pl.pallas_call(
        _logits_body,
        in_specs=[pl.BlockSpec(memory_space=pltpu.HBM),
                  pl.BlockSpec((NUM_EXPERTS, D_MODEL), lambda: (0, 0))],
        out_specs=pl.BlockSpec((NUM_EXPERTS * n,), lambda: (0,)),
        out_shape=jax.ShapeDtypeStruct((NUM_EXPERTS * n,), jnp.float32),
        scratch_shapes=(
            [pltpu.VMEM((_TB, D_MODEL), jnp.float32) for _ in range(_NBUF)]
            + [pltpu.SemaphoreType.DMA((_NBUF,))]
        ),
    )(x, W)


@functools.lru_cache(maxsize=None)
def _make_router(n):
    info = plsc.get_sparse_core_info()
    nc, ns, lanes = info.num_cores, info.num_subcores, info.num_lanes
    nw = nc * ns
    tpw = n // nw
    mesh = plsc.VectorSubcoreMesh(core_axis_name="c", subcore_axis_name="s")

    @functools.partial(
        pl.kernel, mesh=mesh,
        compiler_params=pltpu.CompilerParams(
            needs_layout_passes=False, use_tc_tiling_on_sc=False),
        out_type=[
            jax.ShapeDtypeStruct((TOP_K * n,), jnp.float32),
            jax.ShapeDtypeStruct((TOP_K * n,), jnp.int32),
            jax.ShapeDtypeStruct((MASK_W * n,), jnp.float32),
        ],
        scratch_types=(
            [pltpu.VMEM((tpw,), jnp.float32) for _ in range(NUM_EXPERTS)]
            + [pltpu.VMEM((tpw,), jnp.float32) for _ in range(TOP_K)]
            + [pltpu.VMEM((tpw,), jnp.int32) for _ in range(TOP_K)]
            + [pltpu.VMEM((tpw,), jnp.float32) for _ in range(MASK_W)]
            + [pltpu.SemaphoreType.DMA]
        ),
    )
    def router(logits_hbm, probs_hbm, idx_hbm, mask_hbm, *scratch):
        e_v = scratch[0:NUM_EXPERTS]
        p_v = scratch[NUM_EXPERTS:NUM_EXPERTS + TOP_K]
        ix_v = scratch[NUM_EXPERTS + TOP_K:NUM_EXPERTS + 2 * TOP_K]
        m_v = scratch[NUM_EXPERTS + 2 * TOP_K:NUM_EXPERTS + 2 * TOP_K + MASK_W]
        sem = scratch[-1]
        wid = lax.axis_index("s") * nc + lax.axis_index("c")
        base = wid * tpw
        copies = [
            pltpu.async_copy(
                logits_hbm.at[pl.ds(j * n + base, tpw)], e_v[j], sem)
            for j in range(NUM_EXPERTS)]
        for c in copies:
            c.wait()

        @plsc.parallel_loop(0, tpw // lanes, unroll=2)
        def chunk(i):
            sl = pl.ds(i * lanes, lanes)
            e = [e_v[j][sl] for j in range(NUM_EXPERTS)]
            m1 = e[0]
            i1 = jnp.zeros((lanes,), jnp.int32)
            for j in range(1, NUM_EXPERTS):
                gt = e[j] > m1
                m1 = jnp.where(gt, e[j], m1)
                i1 = jnp.where(gt, j, i1)
            m2 = jnp.full((lanes,), -3e38, jnp.float32)
            i2 = jnp.zeros((lanes,), jnp.int32)
            for j in range(NUM_EXPERTS):
                gt = (e[j] > m2) & (i1 != j)
                m2 = jnp.where(gt, e[j], m2)
                i2 = jnp.where(gt, j, i2)
            d = jnp.exp(m2 - m1)
            p1 = 1.0 / (1.0 + d)
            p2 = d * p1
            p_v[0][sl] = p1
            p_v[1][sl] = p2
            ix_v[0][sl] = i1
            ix_v[1][sl] = i2
            for k in range(NUM_EXPERTS):
                m_v[k][sl] = jnp.where(i1 == k, 1.0, 0.0)
                m_v[NUM_EXPERTS + k][sl] = jnp.where(i2 == k, 1.0, 0.0)

        out = []
        for r in range(TOP_K):
            out.append(pltpu.async_copy(
                p_v[r], probs_hbm.at[pl.ds(r * n + base, tpw)], sem))
            out.append(pltpu.async_copy(
                ix_v[r], idx_hbm.at[pl.ds(r * n + base, tpw)], sem))
        for k in range(MASK_W):
            out.append(pltpu.async_copy(
                m_v[k], mask_hbm.at[pl.ds(k * n + base, tpw)], sem))
        for c in out:
            c.wait()

    return router


def kernel(x, W):
    n = x.shape[0]
    logits_t = _compute_logits_t(x, W)
    probs_t, idx_t, mask_t = _make_router(n)(logits_t)
    probs = probs_t.reshape(TOP_K, n).T
    idx = idx_t.reshape(TOP_K, n).T
    mask = mask_t.reshape(TOP_K, NUM_EXPERTS, n).transpose(2, 0, 1)
    return probs, idx, mask
